# baseline (speedup 1.0000x reference)
import functools

import jax
import jax.numpy as jnp
from jax.experimental import pallas as pl
from jax.experimental.pallas import tpu as pltpu

N = 10000
D_IN = 128
D_OUT = 32
V = 2
BM = 400


def _mlp_kernel(seq_ref, w_ref, b_ref, out_ref):
    out_ref[...] = (
        jnp.dot(seq_ref[...], w_ref[...].T, preferred_element_type=jnp.float32)
        + b_ref[...]
    )


def _agg_kernel(adj_ref, h_ref, hp_ref, fus_ref):
    v = pl.program_id(1)
    hp = jnp.dot(adj_ref[0], h_ref[...], preferred_element_type=jnp.float32)
    hp_ref[0] = hp

    @pl.when(v == 0)
    def _():
        fus_ref[...] = hp * (1.0 / V)

    @pl.when(v != 0)
    def _():
        fus_ref[...] += hp * (1.0 / V)


@jax.jit
def kernel(seq_a, adj_list, W, b):
    b2 = b.reshape(1, D_OUT)
    h_a = pl.pallas_call(
        _mlp_kernel,
        out_shape=jax.ShapeDtypeStruct((N, D_OUT), jnp.float32),
    )(seq_a, W, b2)

    grid = (N // BM, V)
    h_p_list, h_p_fusion = pl.pallas_call(
        _agg_kernel,
        grid=grid,
        in_specs=[
            pl.BlockSpec((1, BM, N), lambda m, v: (v, m, 0)),
            pl.BlockSpec((N, D_OUT), lambda m, v: (0, 0)),
        ],
        out_specs=[
            pl.BlockSpec((1, BM, D_OUT), lambda m, v: (v, m, 0)),
            pl.BlockSpec((BM, D_OUT), lambda m, v: (m, 0)),
        ],
        out_shape=[
            jax.ShapeDtypeStruct((V, N, D_OUT), jnp.float32),
            jax.ShapeDtypeStruct((N, D_OUT), jnp.float32),
        ],
        compiler_params=pltpu.CompilerParams(
            dimension_semantics=("arbitrary", "arbitrary"),
        ),
    )(adj_list, h_a)

    return (h_a, h_p_list, h_p_fusion)


# speedup vs baseline: 1.0171x; 1.0171x over previous
import jax
import jax.numpy as jnp
from jax.experimental import pallas as pl
from jax.experimental.pallas import tpu as pltpu

N = 10000
D_IN = 128
D_OUT = 32
V = 2
BM = 200


def _fused_kernel(seq_ref, w_ref, b_ref, adj_ref, ha_ref, hp_ref, fus_ref,
                  h_scratch):
    m = pl.program_id(0)

    @pl.when(m == 0)
    def _():
        h = (
            jnp.dot(seq_ref[...], w_ref[...].T,
                    preferred_element_type=jnp.float32)
            + b_ref[...]
        )
        h_scratch[...] = h

    m0 = m * BM
    ha_ref[...] = h_scratch[pl.ds(m0, BM), :]
    h = h_scratch[...]
    hp0 = jnp.dot(adj_ref[0], h, preferred_element_type=jnp.float32)
    hp1 = jnp.dot(adj_ref[1], h, preferred_element_type=jnp.float32)
    hp_ref[0] = hp0
    hp_ref[1] = hp1
    fus_ref[...] = (hp0 + hp1) * (1.0 / V)


@jax.jit
def kernel(seq_a, adj_list, W, b):
    b2 = b.reshape(1, D_OUT)
    h_a, h_p_list, h_p_fusion = pl.pallas_call(
        _fused_kernel,
        grid=(N // BM,),
        in_specs=[
            pl.BlockSpec((N, D_IN), lambda m: (0, 0)),
            pl.BlockSpec((D_OUT, D_IN), lambda m: (0, 0)),
            pl.BlockSpec((1, D_OUT), lambda m: (0, 0)),
            pl.BlockSpec((V, BM, N), lambda m: (0, m, 0)),
        ],
        out_specs=[
            pl.BlockSpec((BM, D_OUT), lambda m: (m, 0)),
            pl.BlockSpec((V, BM, D_OUT), lambda m: (0, m, 0)),
            pl.BlockSpec((BM, D_OUT), lambda m: (m, 0)),
        ],
        out_shape=[
            jax.ShapeDtypeStruct((N, D_OUT), jnp.float32),
            jax.ShapeDtypeStruct((V, N, D_OUT), jnp.float32),
            jax.ShapeDtypeStruct((N, D_OUT), jnp.float32),
        ],
        scratch_shapes=[pltpu.VMEM((N, D_OUT), jnp.float32)],
        compiler_params=pltpu.CompilerParams(
            dimension_semantics=("arbitrary",),
        ),
    )(seq_a, W, b2, adj_list)

    return (h_a, h_p_list, h_p_fusion)
